# baseline (speedup 1.0000x reference)
import jax
import jax.numpy as jnp
from jax import lax
from jax.experimental import pallas as pl
from jax.experimental.pallas import tpu as pltpu
from jax.experimental.pallas import tpu_sc as plsc

NIN = 4096
NOUT = 4096
DEPTH = 12
N_NODES = 2**DEPTH - 1
NPAD = 4096
B = 4096
HALF = NOUT // 2

NW = 32
TPW = B // NW

_DN_NT = (((1,), (1,)), ((), ()))


def _mm1_body(x_ref, w_ref, o_ref):
    o_ref[...] = jax.lax.dot_general(
        x_ref[...], w_ref[...], _DN_NT, preferred_element_type=jnp.float32)


def _mm1(x, w1s, bm=1024, bn=1024):
    return pl.pallas_call(
        _mm1_body,
        grid=(B // bm, NPAD // bn),
        in_specs=[
            pl.BlockSpec((bm, NIN), lambda i, j: (i, 0)),
            pl.BlockSpec((bn, NIN), lambda i, j: (j, 0)),
        ],
        out_specs=pl.BlockSpec((bm, bn), lambda i, j: (i, j)),
        out_shape=jax.ShapeDtypeStruct((B, NPAD), jnp.float32),
        compiler_params=pltpu.CompilerParams(
            dimension_semantics=("parallel", "parallel")),
    )(x, w1s)


def _windows():
    wins = []
    for d in range(DEPTH):
        first, last = 2**d - 1, 2**(d + 1) - 2
        lo = (first // 128) * 128
        hi = min(NPAD, (last // 128 + 1) * 128)
        wins.append((lo, hi))
    return wins


_WINS = _windows()


def _walk_body(l_ref, ids_ref, wts_ref):
    br = l_ref.shape[0]
    wts_ref[...] = jnp.zeros_like(wts_ref)
    cur = jnp.zeros((br, 1), jnp.int32)
    for d in range(DEPTH):
        lo, hi = _WINS[d]
        lw = l_ref[:, lo:hi]
        lane = lo + jax.lax.broadcasted_iota(jnp.int32, (br, hi - lo), 1)
        onehot = lane == cur
        sel = jnp.sum(jnp.where(onehot, lw, 0.0), axis=1, keepdims=True)
        ids_ref[:, 2 * d:2 * d + 1] = 2 * cur
        ids_ref[:, 2 * d + 1:2 * d + 2] = 2 * cur + 1
        act = jax.nn.gelu(sel)
        wts_ref[:, 16 * d:16 * (d + 1)] = jnp.broadcast_to(act, (br, 16))
        cur = 2 * cur + 1 + (sel > 0).astype(jnp.int32)


def _walk(l, br=256):
    return pl.pallas_call(
        _walk_body,
        grid=(B // br,),
        in_specs=[pl.BlockSpec((br, NPAD), lambda i: (i, 0))],
        out_specs=[
            pl.BlockSpec((br, 32), lambda i: (i, 0)),
            pl.BlockSpec((br, 256), lambda i: (i, 0)),
        ],
        out_shape=[
            jax.ShapeDtypeStruct((B, 32), jnp.int32),
            jax.ShapeDtypeStruct((B, 256), jnp.float32),
        ],
        compiler_params=pltpu.CompilerParams(
            dimension_semantics=("parallel",)),
    )(l)


def _bag_body(ids_hbm, wts_hbm, w2_hbm, y_hbm,
              ids_v, wts_v, rows_a, rows_b, out_a, out_b, g0, g1, o0, o1):
    wid = lax.axis_index("s") * 2 + lax.axis_index("c")
    base = pl.multiple_of(wid * TPW, TPW)
    pltpu.sync_copy(ids_hbm.at[pl.ds(base * 32, TPW * 32)], ids_v)
    pltpu.sync_copy(wts_hbm.at[pl.ds(base, TPW), :], wts_v)
    rows = (rows_a, rows_b)
    outs = (out_a, out_b)
    gsems = (g0, g1)
    osems = (o0, o1)
    QUART = HALF // 2

    def _gather(t, sl):
        pltpu.async_copy(
            w2_hbm.at[ids_v.at[pl.ds(t * 32, 2 * DEPTH)]],
            rows[sl].at[pl.ds(0, 2 * DEPTH)], gsems[sl])

    _gather(0, 0)
    _gather(1, 1)

    def _token(t, k, pb):
        sl = k % 2
        pltpu.make_async_copy(
            w2_hbm.at[ids_v.at[pl.ds(t * 32, 2 * DEPTH)]],
            rows[sl].at[pl.ds(0, 2 * DEPTH)], gsems[sl]).wait()
        ws = [wts_v[t, pl.ds(16 * j, 16)] for j in range(DEPTH)]
        fmask = jnp.uint32(0xFFFF0000)
        rnd = jnp.uint32(0x7FFF)
        one = jnp.uint32(1)

        def _chunk(c, carry):
            off = pl.multiple_of(c * 16, 16)
            for h in range(2):
                alo = jnp.zeros((16,), jnp.float32)
                ahi = jnp.zeros((16,), jnp.float32)
                for j in range(DEPTH):
                    bits = lax.bitcast_convert_type(
                        rows[sl][2 * j + h, pl.ds(off, 16)], jnp.uint32)
                    lo = lax.bitcast_convert_type(bits << 16, jnp.float32)
                    hi = lax.bitcast_convert_type(bits & fmask, jnp.float32)
                    alo = alo + ws[j] * lo
                    ahi = ahi + ws[j] * hi
                blo = lax.bitcast_convert_type(alo, jnp.uint32)
                bhi = lax.bitcast_convert_type(ahi, jnp.uint32)
                blo = blo + rnd + ((blo >> 16) & one)
                bhi = bhi + rnd + ((bhi >> 16) & one)
                word = (blo >> 16) | (bhi & fmask)
                outs[pb][k, pl.ds(h * QUART + off, 16)] = (
                    lax.bitcast_convert_type(word, jnp.float32))
            return carry

        lax.fori_loop(0, QUART // 16, _chunk, 0)

        @pl.when(t + 2 < TPW)
        def _():
            _gather(t + 2, sl)

    def _block(pb, blk):
        rowbase = pl.multiple_of(base + blk * 8, 8)

        @pl.when(blk >= 2)
        def _():
            pltpu.make_async_copy(
                outs[pb], y_hbm.at[pl.ds(rowbase, 8), :], osems[pb]).wait()

        for k in range(8):
            _token(blk * 8 + k, k, pb)
        pltpu.async_copy(
            outs[pb], y_hbm.at[pl.ds(rowbase, 8), :], osems[pb])

    def _super(b2, carry):
        _block(0, b2 * 2)
        _block(1, b2 * 2 + 1)
        return carry

    lax.fori_loop(0, TPW // 16, _super, 0)
    for pb in range(2):
        blk = TPW // 8 - 2 + pb
        pltpu.make_async_copy(
            outs[pb],
            y_hbm.at[pl.ds(pl.multiple_of(base + blk * 8, 8), 8), :],
            osems[pb]).wait()


def _bag(ids, wts, w2h):
    mesh = plsc.VectorSubcoreMesh(core_axis_name="c", subcore_axis_name="s")
    run = pl.kernel(
        _bag_body,
        out_type=jax.ShapeDtypeStruct((B, HALF), jnp.float32),
        mesh=mesh,
        scratch_types=[
            pltpu.VMEM((B // NW * 32,), jnp.int32),
            pltpu.VMEM((B // NW, 256), jnp.float32),
            pltpu.VMEM((2 * DEPTH, HALF // 2), jnp.float32),
            pltpu.VMEM((2 * DEPTH, HALF // 2), jnp.float32),
            pltpu.VMEM((8, HALF), jnp.float32),
            pltpu.VMEM((8, HALF), jnp.float32),
            pltpu.SemaphoreType.DMA,
            pltpu.SemaphoreType.DMA,
            pltpu.SemaphoreType.DMA,
            pltpu.SemaphoreType.DMA,
        ],
    )
    return run(ids, wts, w2h)


@jax.jit
def kernel(input, w1s, w2s):
    logits = _mm1(input, w1s)
    ids, wts = _walk(logits)
    w2h = jax.lax.bitcast_convert_type(
        w2s.reshape(N_NODES, HALF, 2), jnp.float32).reshape(
            N_NODES * 2, HALF // 2)
    y = _bag(ids.reshape(B * 32), wts, w2h)
    return jax.lax.bitcast_convert_type(y, jnp.bfloat16).reshape(B, NOUT)
